# baseline (speedup 1.0000x reference)
import jax
import jax.numpy as jnp
from jax.experimental import pallas as pl
from jax.experimental.pallas import tpu as pltpu

JOINTS = 22
EMB = 2048
NC = 8
SPLIT = 2
WINDOW = 16
NDMA = NC * SPLIT
RQ = EMB // NC
RD = RQ // SPLIT


def _edge_enhance_kernel(src_ref, b_ref, w_hbm, out_hbm, w_vmem, out_vmem,
                         in_sems, out_sems):
    copies = []
    for d in range(NDMA):
        cp = pltpu.make_async_copy(
            w_hbm.at[pl.ds(d * RD, RD), :],
            w_vmem.at[d // SPLIT, pl.ds((d % SPLIT) * RD, RD), :],
            in_sems.at[d])
        copies.append(cp)
    for d in range(WINDOW):
        copies[d].start()
    src = src_ref[0]
    stores = []
    for q in range(NC):
        for s in range(SPLIT):
            copies[q * SPLIT + s].wait()
            nxt = q * SPLIT + s + WINDOW
            if nxt < NDMA:
                copies[nxt].start()
        y = jax.lax.dot_general(
            src, w_vmem[q],
            dimension_numbers=(((1,), (1,)), ((), ())),
            preferred_element_type=jnp.float32,
        )
        yb = y + b_ref[:, q * RQ:(q + 1) * RQ]
        out_vmem[q] = yb[None, :, :] - y[:, None, :]
        st = pltpu.make_async_copy(
            out_vmem.at[q],
            out_hbm.at[:, :, pl.ds(q * RQ, RQ)],
            out_sems.at[q])
        st.start()
        stores.append(st)
    for st in stores:
        st.wait()


def kernel(src, W, b):
    out = pl.pallas_call(
        _edge_enhance_kernel,
        in_specs=[
            pl.BlockSpec((1, JOINTS, EMB), lambda: (0, 0, 0)),
            pl.BlockSpec((1, EMB), lambda: (0, 0)),
            pl.BlockSpec(memory_space=pltpu.MemorySpace.HBM),
        ],
        out_specs=pl.BlockSpec(memory_space=pltpu.MemorySpace.HBM),
        out_shape=jax.ShapeDtypeStruct((JOINTS, JOINTS, EMB), jnp.float32),
        scratch_shapes=[
            pltpu.VMEM((NC, RQ, EMB), jnp.float32),
            pltpu.VMEM((NC, JOINTS, JOINTS, RQ), jnp.float32),
            pltpu.SemaphoreType.DMA((NDMA,)),
            pltpu.SemaphoreType.DMA((NC,)),
        ],
    )(src, b.reshape(1, EMB), W)
    return out


# speedup vs baseline: 1.0145x; 1.0145x over previous
import jax
import jax.numpy as jnp
from jax.experimental import pallas as pl
from jax.experimental.pallas import tpu as pltpu

JOINTS = 22
EMB = 2048
NC = 4
SPLIT = 2
NDMA = NC * SPLIT
RQ = EMB // NC
RD = RQ // SPLIT


def _edge_enhance_kernel(src_ref, b_ref, w_hbm, out_hbm, w_vmem, out_vmem,
                         in_sems, out_sems):
    copies = []
    for d in range(NDMA):
        cp = pltpu.make_async_copy(
            w_hbm.at[pl.ds(d * RD, RD), :],
            w_vmem.at[d // SPLIT, pl.ds((d % SPLIT) * RD, RD), :],
            in_sems.at[d])
        cp.start()
        copies.append(cp)
    src = src_ref[0]
    stores = []
    for q in range(NC):
        for s in range(SPLIT):
            copies[q * SPLIT + s].wait()
        y = jax.lax.dot_general(
            src, w_vmem[q],
            dimension_numbers=(((1,), (1,)), ((), ())),
            preferred_element_type=jnp.float32,
        )
        yb = y + b_ref[:, q * RQ:(q + 1) * RQ]
        out_vmem[q] = yb[None, :, :] - y[:, None, :]
        st = pltpu.make_async_copy(
            out_vmem.at[q],
            out_hbm.at[:, :, pl.ds(q * RQ, RQ)],
            out_sems.at[q])
        st.start()
        stores.append(st)
    for st in stores:
        st.wait()


def kernel(src, W, b):
    out = pl.pallas_call(
        _edge_enhance_kernel,
        in_specs=[
            pl.BlockSpec((1, JOINTS, EMB), lambda: (0, 0, 0)),
            pl.BlockSpec((1, EMB), lambda: (0, 0)),
            pl.BlockSpec(memory_space=pltpu.MemorySpace.HBM),
        ],
        out_specs=pl.BlockSpec(memory_space=pltpu.MemorySpace.HBM),
        out_shape=jax.ShapeDtypeStruct((JOINTS, JOINTS, EMB), jnp.float32),
        scratch_shapes=[
            pltpu.VMEM((NC, RQ, EMB), jnp.float32),
            pltpu.VMEM((NC, JOINTS, JOINTS, RQ), jnp.float32),
            pltpu.SemaphoreType.DMA((NDMA,)),
            pltpu.SemaphoreType.DMA((NC,)),
        ],
    )(src, b.reshape(1, EMB), W)
    return out
